# Initial kernel scaffold; baseline (speedup 1.0000x reference)
import jax
import jax.numpy as jnp
from jax.experimental import pallas as pl


def kernel(inputs, categories, mask_positions, tokens_embedding):
    raise NotImplementedError("write your pallas kernel here")


# speedup vs baseline: 5.6857x; 5.6857x over previous
import functools

import jax
import jax.numpy as jnp
from jax import lax
from jax.experimental import pallas as pl
from jax.experimental.pallas import tpu as pltpu
from jax.experimental.pallas import tpu_sc as plsc


def _copy_body(x_ref, o_ref):
    o_ref[...] = x_ref[...]


def _bulk_copy(flat):
    B, W = flat.shape
    bb = 16
    return pl.pallas_call(
        _copy_body,
        grid=(B // bb,),
        in_specs=[pl.BlockSpec((bb, W), lambda i: (i, 0))],
        out_specs=pl.BlockSpec((bb, W), lambda i: (i, 0)),
        out_shape=jax.ShapeDtypeStruct((B, W), jnp.float32),
    )(flat)


def _make_sc_scatter(B, L, H, M, V):
    info = plsc.get_sparse_core_info()
    NC, NS, LN = info.num_cores, info.num_subcores, info.num_lanes
    NW = NC * NS
    bpw = B // NW
    ipw = bpw * M
    CH = 128
    NCH = ipw // CH
    UNROLL = CH // LN

    mesh = plsc.VectorSubcoreMesh(core_axis_name="c", subcore_axis_name="s")

    @functools.partial(
        pl.kernel,
        mesh=mesh,
        out_type=(),
        compiler_params=pltpu.CompilerParams(
            use_tc_tiling_on_sc=False, needs_layout_passes=False),
        scratch_types=[
            pltpu.VMEM((ipw,), jnp.int32),
            pltpu.VMEM((bpw, L), jnp.int32),
            pltpu.VMEM((NCH, CH), jnp.int32),
            pltpu.VMEM((NCH, CH), jnp.int32),
            pltpu.VMEM((CH, H), jnp.float32),
            pltpu.SemaphoreType.DMA,
        ],
    )
    def sc_scatter(out_hbm, cat_hbm, pos_hbm, emb_hbm,
                   pos_v, cat_v, tok_v, ridx_v, rows_v, sem):
        wid = lax.axis_index("s") * NC + lax.axis_index("c")
        b0 = wid * bpw
        j0 = wid * ipw
        pltpu.sync_copy(pos_hbm.at[pl.ds(j0, ipw)], pos_v)
        pltpu.sync_copy(cat_hbm.at[pl.ds(b0, bpw)], cat_v)

        @pl.loop(0, NCH)
        def _chunk(ci):
            for u in range(UNROLL):
                o = ci * CH + u * LN
                pv = pos_v[pl.ds(o, LN)]
                jj = o + lax.iota(jnp.int32, LN)
                bl = lax.div(jj, M)
                tok = plsc.load_gather(cat_v, [bl, pv])
                tok_v[ci, pl.ds(u * LN, LN)] = tok
                ridx_v[ci, pl.ds(u * LN, LN)] = (b0 + bl) * L + pv
            pltpu.async_copy(emb_hbm.at[tok_v.at[ci]], rows_v, sem).wait()
            pltpu.async_copy(rows_v, out_hbm.at[ridx_v.at[ci]], sem).wait()

    return sc_scatter


def kernel(inputs, categories, mask_positions, tokens_embedding):
    B, L, H = inputs.shape
    M = mask_positions.shape[1]
    V = tokens_embedding.shape[0]

    pos_flat = mask_positions.reshape(B * M)

    out2 = _bulk_copy(inputs.reshape(B, L * H))
    out_ref = jax.new_ref(out2.reshape(B * L, H))

    sc_scatter = _make_sc_scatter(B, L, H, M, V)
    sc_scatter(out_ref, categories, pos_flat, tokens_embedding)

    return out_ref[...].reshape(B, L, H)
